# baseline (speedup 1.0000x reference)
import functools

import numpy as np
import jax
import jax.numpy as jnp
from jax import lax
from jax.experimental import pallas as pl
from jax.experimental.pallas import tpu as pltpu
from jax.experimental.pallas import tpu_sc as plsc

LANE = 128
NSC = 2
NTILE = 16
NW = NSC * NTILE
NB = 80
EP = NW * NB * LANE
NP = 10240
PER_T = NP // NTILE
ZR = 128
BR = 1000


def _deg_body(dstp_hbm, outdeg_hbm, didx, ones_v, zvec, deg_sh, sem_d):
    cid = lax.axis_index("c")
    sid = lax.axis_index("s")
    wid = cid * NTILE + sid
    zero = jnp.zeros((16,), jnp.float32)
    one = jnp.ones((16,), jnp.float32)
    for i in range(PER_T // 16):
        zvec[pl.ds(i * 16, 16)] = zero
    for i in range(LANE // 16):
        ones_v[pl.ds(i * 16, 16)] = one
    pltpu.sync_copy(zvec, deg_sh.at[pl.ds(sid * PER_T, PER_T)])
    pltpu.sync_copy(dstp_hbm.at[pl.ds(wid * NB, NB)], didx)
    plsc.subcore_barrier()

    def issue(j, c):
        pltpu.async_copy(ones_v, deg_sh.at[didx.at[j]], sem_d, add=True)
        return c

    lax.fori_loop(0, NB, issue, 0)

    def drain(j, c):
        pltpu.make_async_copy(ones_v, deg_sh.at[didx.at[0]], sem_d).wait()
        return c

    lax.fori_loop(0, NB, drain, 0)
    plsc.subcore_barrier()

    @pl.when(sid == 0)
    def _():
        pltpu.sync_copy(deg_sh, outdeg_hbm.at[cid])


def _scat_body(g_hbm, srcp_hbm, dstp_hbm, out_hbm,
               sidx, dsa, dsb, bufa, bufb, s_sh,
               sem_a, sem_b, sem_da, sem_db):
    cid = lax.axis_index("c")
    sid = lax.axis_index("s")
    wid = cid * NTILE + sid
    base = wid * NB
    zero = jnp.zeros((16,), jnp.float32)

    def zrow(i, c):
        for k in range(LANE // 16):
            bufa[i, pl.ds(k * 16, 16)] = zero
        return c

    lax.fori_loop(0, ZR, zrow, 0)
    for k in range(PER_T // ZR):
        pltpu.sync_copy(bufa, s_sh.at[pl.ds(sid * PER_T + k * ZR, ZR)])

    pltpu.sync_copy(srcp_hbm.at[pl.ds(base, NB)], sidx)
    pltpu.async_copy(dstp_hbm.at[pl.ds(base, 2)], dsa, sem_da)
    pltpu.async_copy(dstp_hbm.at[pl.ds(base + 2, 2)], dsb, sem_db)
    pltpu.async_copy(g_hbm.at[sidx.at[0]], bufa, sem_a)
    plsc.subcore_barrier()

    def pair(q0, dslot, sem_d):
        q1 = q0 + 1
        pltpu.make_async_copy(dstp_hbm.at[pl.ds(base, 2)], dslot,
                              sem_d).wait()
        pltpu.make_async_copy(g_hbm.at[sidx.at[q0]], bufa, sem_a).wait()
        pltpu.async_copy(g_hbm.at[sidx.at[q1]], bufb, sem_b)
        pltpu.sync_copy(bufa, s_sh.at[dslot.at[0]], add=True)
        pltpu.make_async_copy(g_hbm.at[sidx.at[q1]], bufb, sem_b).wait()
        nxt = jnp.minimum(q1 + 1, NB - 1)
        pltpu.async_copy(g_hbm.at[sidx.at[nxt]], bufa, sem_a)
        pltpu.sync_copy(bufb, s_sh.at[dslot.at[1]], add=True)
        pre = jnp.minimum(q0 + 4, NB - 2)
        pltpu.async_copy(dstp_hbm.at[pl.ds(base + pre, 2)], dslot, sem_d)

    def body(j, c):
        q = 4 * j
        pair(q, dsa, sem_da)
        pair(q + 2, dsb, sem_db)
        return c

    lax.fori_loop(0, NB // 4, body, 0)
    pltpu.make_async_copy(g_hbm.at[sidx.at[NB - 1]], bufa, sem_a).wait()
    pltpu.make_async_copy(dstp_hbm.at[pl.ds(base, 2)], dsa, sem_da).wait()
    pltpu.make_async_copy(dstp_hbm.at[pl.ds(base, 2)], dsb, sem_db).wait()
    plsc.subcore_barrier()
    pltpu.sync_copy(s_sh.at[pl.ds(sid * PER_T, PER_T)],
                    out_hbm.at[cid, pl.ds(sid * PER_T, PER_T)])


def _h_body(x_ref, w_ref, o_ref):
    o_ref[...] = lax.dot_general(x_ref[...], w_ref[...],
                                 (((1,), (1,)), ((), ())),
                                 preferred_element_type=jnp.float32)


def _g_body(h_ref, p0_ref, p1_ref, o_ref):
    dinv = lax.rsqrt(p0_ref[...] + p1_ref[...] + 1.0)
    o_ref[...] = h_ref[...] * dinv


def _fin_body(x_ref, s_ref, g_ref, p0_ref, p1_ref, b_ref, a_ref, o_ref):
    dinv = lax.rsqrt(p0_ref[...] + p1_ref[...] + 1.0)
    a = a_ref[0, 0]
    aggr = dinv * (s_ref[0] + s_ref[1] + g_ref[...]) + b_ref[...]
    o_ref[...] = a * x_ref[...] + (1.0 - a) * aggr


def kernel(node_features, edge_index, W, b, alpha):
    x = node_features
    n, d = x.shape
    e = edge_index.shape[1]

    pad = EP - e
    pidx = np.arange(pad, dtype=np.int32)
    pad_src = jnp.asarray((pidx % n).reshape(-1, LANE))
    pad_dst = jnp.asarray((n + (pidx % 16)).reshape(-1, LANE))
    ei3 = edge_index.reshape(2, e // LANE, LANE)
    src_p = jnp.concatenate([ei3[0], pad_src], axis=0)
    dst_p = jnp.concatenate([ei3[1], pad_dst], axis=0)

    mesh = plsc.VectorSubcoreMesh(core_axis_name="c", subcore_axis_name="s")

    deg_parts = pl.kernel(
        _deg_body,
        out_type=jax.ShapeDtypeStruct((NSC, NP), jnp.float32),
        mesh=mesh,
        scratch_types=[
            pltpu.VMEM((NB, LANE), jnp.int32),
            pltpu.VMEM((LANE,), jnp.float32),
            pltpu.VMEM((PER_T,), jnp.float32),
            pltpu.VMEM_SHARED((NP,), jnp.float32),
            pltpu.SemaphoreType.DMA,
        ],
    )(dst_p)

    p0 = deg_parts[0, :n, None]
    p1 = deg_parts[1, :n, None]

    h = pl.pallas_call(
        _h_body,
        grid=(n // BR,),
        in_specs=[
            pl.BlockSpec((BR, d), lambda i: (i, 0)),
            pl.BlockSpec((d, d), lambda i: (0, 0)),
        ],
        out_specs=pl.BlockSpec((BR, d), lambda i: (i, 0)),
        out_shape=jax.ShapeDtypeStruct((n, d), jnp.float32),
    )(x, W)

    g = pl.pallas_call(
        _g_body,
        grid=(n // BR,),
        in_specs=[
            pl.BlockSpec((BR, d), lambda i: (i, 0)),
            pl.BlockSpec((BR, 1), lambda i: (i, 0)),
            pl.BlockSpec((BR, 1), lambda i: (i, 0)),
        ],
        out_specs=pl.BlockSpec((BR, d), lambda i: (i, 0)),
        out_shape=jax.ShapeDtypeStruct((n, d), jnp.float32),
    )(h, p0, p1)

    s_parts = pl.kernel(
        _scat_body,
        out_type=jax.ShapeDtypeStruct((NSC, NP, d), jnp.float32),
        mesh=mesh,
        scratch_types=[
            pltpu.VMEM((NB, LANE), jnp.int32),
            pltpu.VMEM((2, LANE), jnp.int32),
            pltpu.VMEM((2, LANE), jnp.int32),
            pltpu.VMEM((LANE, d), jnp.float32),
            pltpu.VMEM((LANE, d), jnp.float32),
            pltpu.VMEM_SHARED((NP, d), jnp.float32),
            pltpu.SemaphoreType.DMA,
            pltpu.SemaphoreType.DMA,
            pltpu.SemaphoreType.DMA,
            pltpu.SemaphoreType.DMA,
        ],
    )(g, src_p, dst_p)

    out = pl.pallas_call(
        _fin_body,
        grid=(n // BR,),
        in_specs=[
            pl.BlockSpec((BR, d), lambda i: (i, 0)),
            pl.BlockSpec((NSC, BR, d), lambda i: (0, i, 0)),
            pl.BlockSpec((BR, d), lambda i: (i, 0)),
            pl.BlockSpec((BR, 1), lambda i: (i, 0)),
            pl.BlockSpec((BR, 1), lambda i: (i, 0)),
            pl.BlockSpec((1, d), lambda i: (0, 0)),
            pl.BlockSpec((1, 1), lambda i: (0, 0)),
        ],
        out_specs=pl.BlockSpec((BR, d), lambda i: (i, 0)),
        out_shape=jax.ShapeDtypeStruct((n, d), jnp.float32),
    )(x, s_parts, g, p0, p1, b.reshape(1, d), alpha.reshape(1, 1))
    return out


# speedup vs baseline: 1.0304x; 1.0304x over previous
import functools

import numpy as np
import jax
import jax.numpy as jnp
from jax import lax
from jax.experimental import pallas as pl
from jax.experimental.pallas import tpu as pltpu
from jax.experimental.pallas import tpu_sc as plsc

LANE = 128
NSC = 2
NTILE = 16
NW = NSC * NTILE
NB = 80
EP = NW * NB * LANE
NP = 10240
PER_T = NP // NTILE
ZR = 128
BR = 1000


def _deg_body(dstp_hbm, outdeg_hbm, didx, ones_v, zvec, deg_sh, sem_d):
    cid = lax.axis_index("c")
    sid = lax.axis_index("s")
    wid = cid * NTILE + sid
    zero = jnp.zeros((16,), jnp.float32)
    one = jnp.ones((16,), jnp.float32)
    for i in range(PER_T // 16):
        zvec[pl.ds(i * 16, 16)] = zero
    for i in range(LANE // 16):
        ones_v[pl.ds(i * 16, 16)] = one
    pltpu.sync_copy(zvec, deg_sh.at[pl.ds(sid * PER_T, PER_T)])
    pltpu.sync_copy(dstp_hbm.at[pl.ds(wid * NB, NB)], didx)
    plsc.subcore_barrier()

    def issue(j, c):
        pltpu.async_copy(ones_v, deg_sh.at[didx.at[j]], sem_d, add=True)
        return c

    lax.fori_loop(0, NB, issue, 0)

    def drain(j, c):
        pltpu.make_async_copy(ones_v, deg_sh.at[didx.at[0]], sem_d).wait()
        return c

    lax.fori_loop(0, NB, drain, 0)
    plsc.subcore_barrier()

    @pl.when(sid == 0)
    def _():
        pltpu.sync_copy(deg_sh, outdeg_hbm.at[cid])


def _scat_body(g_hbm, srcp_hbm, dstp_hbm, out_hbm,
               sidx, dsa, dsb, bufa, bufb, s_sh,
               sem_a, sem_b, sem_da, sem_db):
    cid = lax.axis_index("c")
    sid = lax.axis_index("s")
    wid = cid * NTILE + sid
    base = wid * NB
    zero = jnp.zeros((16,), jnp.float32)

    def zrow(i, c):
        for k in range(LANE // 16):
            bufa[i, pl.ds(k * 16, 16)] = zero
        return c

    lax.fori_loop(0, ZR, zrow, 0)
    for k in range(PER_T // ZR):
        pltpu.sync_copy(bufa, s_sh.at[pl.ds(sid * PER_T + k * ZR, ZR)])

    pltpu.sync_copy(srcp_hbm.at[pl.ds(base, NB)], sidx)
    pltpu.async_copy(dstp_hbm.at[pl.ds(base, 2)], dsa, sem_da)
    pltpu.async_copy(dstp_hbm.at[pl.ds(base + 2, 2)], dsb, sem_db)
    pltpu.async_copy(g_hbm.at[sidx.at[0]], bufa, sem_a)
    plsc.subcore_barrier()

    def pair(q0, dslot, sem_d):
        q1 = q0 + 1
        pltpu.make_async_copy(dstp_hbm.at[pl.ds(base, 2)], dslot,
                              sem_d).wait()
        pltpu.make_async_copy(g_hbm.at[sidx.at[q0]], bufa, sem_a).wait()
        pltpu.async_copy(g_hbm.at[sidx.at[q1]], bufb, sem_b)
        pltpu.sync_copy(bufa, s_sh.at[dslot.at[0]], add=True)
        pltpu.make_async_copy(g_hbm.at[sidx.at[q1]], bufb, sem_b).wait()
        nxt = jnp.minimum(q1 + 1, NB - 1)
        pltpu.async_copy(g_hbm.at[sidx.at[nxt]], bufa, sem_a)
        pltpu.sync_copy(bufb, s_sh.at[dslot.at[1]], add=True)
        pre = jnp.minimum(q0 + 4, NB - 2)
        pltpu.async_copy(dstp_hbm.at[pl.ds(base + pre, 2)], dslot, sem_d)

    def body(j, c):
        q = 4 * j
        pair(q, dsa, sem_da)
        pair(q + 2, dsb, sem_db)
        return c

    lax.fori_loop(0, NB // 4, body, 0)
    pltpu.make_async_copy(g_hbm.at[sidx.at[NB - 1]], bufa, sem_a).wait()
    pltpu.make_async_copy(dstp_hbm.at[pl.ds(base, 2)], dsa, sem_da).wait()
    pltpu.make_async_copy(dstp_hbm.at[pl.ds(base, 2)], dsb, sem_db).wait()
    plsc.subcore_barrier()
    pltpu.sync_copy(s_sh.at[pl.ds(sid * PER_T, PER_T)],
                    out_hbm.at[cid, pl.ds(sid * PER_T, PER_T)])


def _g_body(x_ref, w_ref, p0_ref, p1_ref, o_ref):
    dinv = lax.rsqrt(p0_ref[...] + p1_ref[...] + 1.0)
    h = lax.dot_general(x_ref[...], w_ref[...], (((1,), (1,)), ((), ())),
                        preferred_element_type=jnp.float32)
    o_ref[...] = h * dinv


def _fin_body(x_ref, s_ref, g_ref, p0_ref, p1_ref, b_ref, a_ref, o_ref):
    dinv = lax.rsqrt(p0_ref[...] + p1_ref[...] + 1.0)
    a = a_ref[0, 0]
    aggr = dinv * (s_ref[0] + s_ref[1] + g_ref[...]) + b_ref[...]
    o_ref[...] = a * x_ref[...] + (1.0 - a) * aggr


def kernel(node_features, edge_index, W, b, alpha):
    x = node_features
    n, d = x.shape
    e = edge_index.shape[1]

    pad = EP - e
    pidx = np.arange(pad, dtype=np.int32)
    pad_src = jnp.asarray(pidx % n)
    pad_dst = jnp.asarray(n + (pidx % 16))
    src_p = jnp.concatenate([edge_index[0], pad_src]).reshape(NW * NB, LANE)
    dst_p = jnp.concatenate([edge_index[1], pad_dst]).reshape(NW * NB, LANE)

    mesh = plsc.VectorSubcoreMesh(core_axis_name="c", subcore_axis_name="s")

    deg_parts = pl.kernel(
        _deg_body,
        out_type=jax.ShapeDtypeStruct((NSC, NP), jnp.float32),
        mesh=mesh,
        scratch_types=[
            pltpu.VMEM((NB, LANE), jnp.int32),
            pltpu.VMEM((LANE,), jnp.float32),
            pltpu.VMEM((PER_T,), jnp.float32),
            pltpu.VMEM_SHARED((NP,), jnp.float32),
            pltpu.SemaphoreType.DMA,
        ],
    )(dst_p)

    p0 = deg_parts[0, :n, None]
    p1 = deg_parts[1, :n, None]

    g = pl.pallas_call(
        _g_body,
        grid=(n // BR,),
        in_specs=[
            pl.BlockSpec((BR, d), lambda i: (i, 0)),
            pl.BlockSpec((d, d), lambda i: (0, 0)),
            pl.BlockSpec((BR, 1), lambda i: (i, 0)),
            pl.BlockSpec((BR, 1), lambda i: (i, 0)),
        ],
        out_specs=pl.BlockSpec((BR, d), lambda i: (i, 0)),
        out_shape=jax.ShapeDtypeStruct((n, d), jnp.float32),
    )(x, W, p0, p1)

    s_parts = pl.kernel(
        _scat_body,
        out_type=jax.ShapeDtypeStruct((NSC, NP, d), jnp.float32),
        mesh=mesh,
        scratch_types=[
            pltpu.VMEM((NB, LANE), jnp.int32),
            pltpu.VMEM((2, LANE), jnp.int32),
            pltpu.VMEM((2, LANE), jnp.int32),
            pltpu.VMEM((LANE, d), jnp.float32),
            pltpu.VMEM((LANE, d), jnp.float32),
            pltpu.VMEM_SHARED((NP, d), jnp.float32),
            pltpu.SemaphoreType.DMA,
            pltpu.SemaphoreType.DMA,
            pltpu.SemaphoreType.DMA,
            pltpu.SemaphoreType.DMA,
        ],
    )(g, src_p, dst_p)

    out = pl.pallas_call(
        _fin_body,
        grid=(n // BR,),
        in_specs=[
            pl.BlockSpec((BR, d), lambda i: (i, 0)),
            pl.BlockSpec((NSC, BR, d), lambda i: (0, i, 0)),
            pl.BlockSpec((BR, d), lambda i: (i, 0)),
            pl.BlockSpec((BR, 1), lambda i: (i, 0)),
            pl.BlockSpec((BR, 1), lambda i: (i, 0)),
            pl.BlockSpec((1, d), lambda i: (0, 0)),
            pl.BlockSpec((1, 1), lambda i: (0, 0)),
        ],
        out_specs=pl.BlockSpec((BR, d), lambda i: (i, 0)),
        out_shape=jax.ShapeDtypeStruct((n, d), jnp.float32),
    )(x, s_parts, g, p0, p1, b.reshape(1, d), alpha.reshape(1, 1))
    return out


# speedup vs baseline: 1.2834x; 1.2455x over previous
import functools

import numpy as np
import jax
import jax.numpy as jnp
from jax import lax
from jax.experimental import pallas as pl
from jax.experimental.pallas import tpu as pltpu
from jax.experimental.pallas import tpu_sc as plsc

LANE = 128
NSC = 2
NTILE = 16
NW = NSC * NTILE
NBF = 78
XBASE = NW * NBF * LANE
NXW = 4
NP = 10240
PER_T = NP // NTILE
ZR = 128
BR = 1000


def _deg_body(ei_hbm, outdeg_hbm, itix, ones_v, zvec, deg_sh, sem_l, sem_s):
    cid = lax.axis_index("c")
    sid = lax.axis_index("s")
    wid = cid * NTILE + sid
    base = wid * NBF * LANE
    zero = jnp.zeros((16,), jnp.float32)
    one = jnp.ones((16,), jnp.float32)
    for i in range(PER_T // 16):
        zvec[pl.ds(i * 16, 16)] = zero
    for i in range(LANE // 16):
        ones_v[pl.ds(i * 16, 16)] = one

    def load(j, c):
        off = pl.multiple_of(base + j * LANE, LANE)
        pltpu.async_copy(ei_hbm.at[pl.ds(0, 2), pl.ds(off, LANE)],
                         itix.at[j], sem_l)
        return c

    lax.fori_loop(0, NBF, load, 0)

    @pl.when(wid < NXW)
    def _():
        pltpu.async_copy(
            ei_hbm.at[pl.ds(0, 2), pl.ds(XBASE + wid * LANE, LANE)],
            itix.at[NBF], sem_l)

    pltpu.sync_copy(zvec, deg_sh.at[pl.ds(sid * PER_T, PER_T)])

    def load_drain(j, c):
        pltpu.make_async_copy(ei_hbm.at[pl.ds(0, 2), pl.ds(base, LANE)],
                              itix.at[0], sem_l).wait()
        return c

    lax.fori_loop(0, NBF, load_drain, 0)

    @pl.when(wid < NXW)
    def _():
        pltpu.make_async_copy(ei_hbm.at[pl.ds(0, 2), pl.ds(base, LANE)],
                              itix.at[0], sem_l).wait()

    plsc.subcore_barrier()

    def issue(j, c):
        pltpu.async_copy(ones_v, deg_sh.at[itix.at[j, 1]], sem_s, add=True)
        return c

    lax.fori_loop(0, NBF, issue, 0)

    @pl.when(wid < NXW)
    def _():
        pltpu.async_copy(ones_v, deg_sh.at[itix.at[NBF, 1]], sem_s,
                         add=True)

    def drain(j, c):
        pltpu.make_async_copy(ones_v, deg_sh.at[itix.at[0, 1]],
                              sem_s).wait()
        return c

    lax.fori_loop(0, NBF, drain, 0)

    @pl.when(wid < NXW)
    def _():
        pltpu.make_async_copy(ones_v, deg_sh.at[itix.at[0, 1]],
                              sem_s).wait()

    plsc.subcore_barrier()

    @pl.when(sid == 0)
    def _():
        pltpu.sync_copy(deg_sh, outdeg_hbm.at[cid])


def _scat_body(g_hbm, ei_hbm, out_hbm,
               s0, s1, s2, s3, bufa, bufb, s_sh,
               sem_a, sem_b, sem_l0, sem_l1, sem_l2, sem_l3):
    cid = lax.axis_index("c")
    sid = lax.axis_index("s")
    wid = cid * NTILE + sid
    base = wid * NBF * LANE
    zero = jnp.zeros((16,), jnp.float32)

    def tile_at(b):
        off = pl.multiple_of(base + jnp.minimum(b, NBF - 1) * LANE, LANE)
        return ei_hbm.at[pl.ds(0, 2), pl.ds(off, LANE)]

    pltpu.async_copy(tile_at(0), s0, sem_l0)
    pltpu.async_copy(tile_at(1), s1, sem_l1)
    pltpu.async_copy(tile_at(2), s2, sem_l2)
    pltpu.async_copy(tile_at(3), s3, sem_l3)

    def zrow(i, c):
        for k in range(LANE // 16):
            bufa[i, pl.ds(k * 16, 16)] = zero
        return c

    lax.fori_loop(0, ZR, zrow, 0)
    for k in range(PER_T // ZR):
        pltpu.sync_copy(bufa, s_sh.at[pl.ds(sid * PER_T + k * ZR, ZR)])

    pltpu.make_async_copy(tile_at(0), s0, sem_l0).wait()
    pltpu.async_copy(g_hbm.at[s0.at[0]], bufa, sem_a)
    pltpu.make_async_copy(tile_at(1), s1, sem_l1).wait()
    pltpu.async_copy(g_hbm.at[s1.at[0]], bufb, sem_b)
    plsc.subcore_barrier()

    def body(j, c):
        q = 4 * j
        pltpu.make_async_copy(g_hbm.at[s0.at[0]], bufa, sem_a).wait()
        pltpu.sync_copy(bufa, s_sh.at[s0.at[1]], add=True)
        pltpu.make_async_copy(tile_at(q + 2), s2, sem_l2).wait()
        pltpu.async_copy(g_hbm.at[s2.at[0]], bufa, sem_a)
        pltpu.make_async_copy(g_hbm.at[s1.at[0]], bufb, sem_b).wait()
        pltpu.sync_copy(bufb, s_sh.at[s1.at[1]], add=True)
        pltpu.make_async_copy(tile_at(q + 3), s3, sem_l3).wait()
        pltpu.async_copy(g_hbm.at[s3.at[0]], bufb, sem_b)
        pltpu.async_copy(tile_at(q + 4), s0, sem_l0)
        pltpu.async_copy(tile_at(q + 5), s1, sem_l1)
        pltpu.make_async_copy(g_hbm.at[s2.at[0]], bufa, sem_a).wait()
        pltpu.sync_copy(bufa, s_sh.at[s2.at[1]], add=True)
        pltpu.make_async_copy(tile_at(q + 4), s0, sem_l0).wait()
        pltpu.async_copy(g_hbm.at[s0.at[0]], bufa, sem_a)
        pltpu.make_async_copy(g_hbm.at[s3.at[0]], bufb, sem_b).wait()
        pltpu.sync_copy(bufb, s_sh.at[s3.at[1]], add=True)
        pltpu.make_async_copy(tile_at(q + 5), s1, sem_l1).wait()
        pltpu.async_copy(g_hbm.at[s1.at[0]], bufb, sem_b)
        pltpu.async_copy(tile_at(q + 6), s2, sem_l2)
        pltpu.async_copy(tile_at(q + 7), s3, sem_l3)
        return c

    lax.fori_loop(0, (NBF - 2) // 4, body, 0)
    pltpu.make_async_copy(g_hbm.at[s0.at[0]], bufa, sem_a).wait()
    pltpu.sync_copy(bufa, s_sh.at[s0.at[1]], add=True)
    pltpu.make_async_copy(g_hbm.at[s1.at[0]], bufb, sem_b).wait()
    pltpu.sync_copy(bufb, s_sh.at[s1.at[1]], add=True)
    pltpu.make_async_copy(tile_at(0), s2, sem_l2).wait()
    pltpu.make_async_copy(tile_at(0), s3, sem_l3).wait()

    @pl.when(wid < NXW)
    def _():
        pltpu.sync_copy(
            ei_hbm.at[pl.ds(0, 2), pl.ds(XBASE + wid * LANE, LANE)], s2)
        pltpu.sync_copy(g_hbm.at[s2.at[0]], bufa)
        pltpu.sync_copy(bufa, s_sh.at[s2.at[1]], add=True)

    plsc.subcore_barrier()
    pltpu.sync_copy(s_sh.at[pl.ds(sid * PER_T, PER_T)],
                    out_hbm.at[cid, pl.ds(sid * PER_T, PER_T)])


def _g_body(x_ref, w_ref, p0_ref, p1_ref, o_ref):
    dinv = lax.rsqrt(p0_ref[...] + p1_ref[...] + 1.0)
    h = lax.dot_general(x_ref[...], w_ref[...], (((1,), (1,)), ((), ())),
                        preferred_element_type=jnp.float32)
    o_ref[...] = h * dinv


def _fin_body(x_ref, s_ref, g_ref, p0_ref, p1_ref, b_ref, a_ref, o_ref):
    dinv = lax.rsqrt(p0_ref[...] + p1_ref[...] + 1.0)
    a = a_ref[0, 0]
    aggr = dinv * (s_ref[0] + s_ref[1] + g_ref[...]) + b_ref[...]
    o_ref[...] = a * x_ref[...] + (1.0 - a) * aggr


def kernel(node_features, edge_index, W, b, alpha):
    x = node_features
    n, d = x.shape

    mesh = plsc.VectorSubcoreMesh(core_axis_name="c", subcore_axis_name="s")

    deg_parts = pl.kernel(
        _deg_body,
        out_type=jax.ShapeDtypeStruct((NSC, NP), jnp.float32),
        mesh=mesh,
        scratch_types=[
            pltpu.VMEM((NBF + 1, 2, LANE), jnp.int32),
            pltpu.VMEM((LANE,), jnp.float32),
            pltpu.VMEM((PER_T,), jnp.float32),
            pltpu.VMEM_SHARED((NP,), jnp.float32),
            pltpu.SemaphoreType.DMA,
            pltpu.SemaphoreType.DMA,
        ],
    )(edge_index)

    p0 = deg_parts[0, :n, None]
    p1 = deg_parts[1, :n, None]

    g = pl.pallas_call(
        _g_body,
        grid=(n // BR,),
        in_specs=[
            pl.BlockSpec((BR, d), lambda i: (i, 0)),
            pl.BlockSpec((d, d), lambda i: (0, 0)),
            pl.BlockSpec((BR, 1), lambda i: (i, 0)),
            pl.BlockSpec((BR, 1), lambda i: (i, 0)),
        ],
        out_specs=pl.BlockSpec((BR, d), lambda i: (i, 0)),
        out_shape=jax.ShapeDtypeStruct((n, d), jnp.float32),
    )(x, W, p0, p1)

    s_parts = pl.kernel(
        _scat_body,
        out_type=jax.ShapeDtypeStruct((NSC, NP, d), jnp.float32),
        mesh=mesh,
        scratch_types=[
            pltpu.VMEM((2, LANE), jnp.int32),
            pltpu.VMEM((2, LANE), jnp.int32),
            pltpu.VMEM((2, LANE), jnp.int32),
            pltpu.VMEM((2, LANE), jnp.int32),
            pltpu.VMEM((LANE, d), jnp.float32),
            pltpu.VMEM((LANE, d), jnp.float32),
            pltpu.VMEM_SHARED((NP, d), jnp.float32),
            pltpu.SemaphoreType.DMA,
            pltpu.SemaphoreType.DMA,
            pltpu.SemaphoreType.DMA,
            pltpu.SemaphoreType.DMA,
            pltpu.SemaphoreType.DMA,
            pltpu.SemaphoreType.DMA,
        ],
    )(g, edge_index)

    out = pl.pallas_call(
        _fin_body,
        grid=(n // BR,),
        in_specs=[
            pl.BlockSpec((BR, d), lambda i: (i, 0)),
            pl.BlockSpec((NSC, BR, d), lambda i: (0, i, 0)),
            pl.BlockSpec((BR, d), lambda i: (i, 0)),
            pl.BlockSpec((BR, 1), lambda i: (i, 0)),
            pl.BlockSpec((BR, 1), lambda i: (i, 0)),
            pl.BlockSpec((1, d), lambda i: (0, 0)),
            pl.BlockSpec((1, 1), lambda i: (0, 0)),
        ],
        out_specs=pl.BlockSpec((BR, d), lambda i: (i, 0)),
        out_shape=jax.ShapeDtypeStruct((n, d), jnp.float32),
    )(x, s_parts, g, p0, p1, b.reshape(1, d), alpha.reshape(1, 1))
    return out
